# baseline (speedup 1.0000x reference)
import numpy as np
import jax
import jax.numpy as jnp
from jax.experimental import pallas as pl
from jax.experimental.pallas import tpu as pltpu

FEAT_STRIDE = 16
IM_DIMS = (512, 512)
ANCHOR_SCALES = (8, 16, 32)
ANCHOR_RATIOS = (0.5, 1.0, 2.0)
PRE_NMS = 2000
POST_NMS = 300
NMS_THRESH = 0.7
MIN_SIZE = 16.0

N_PAD = 2048
T = 128
K = N_PAD // T
OUT_PAD = 384


def _gen_base_anchors(base_size=16, ratios=ANCHOR_RATIOS, scales=ANCHOR_SCALES):
    base = np.array([0, 0, base_size - 1, base_size - 1], dtype=np.float64)

    def whctrs(a):
        w = a[2] - a[0] + 1.0
        h = a[3] - a[1] + 1.0
        return w, h, a[0] + 0.5 * (w - 1), a[1] + 0.5 * (h - 1)

    def mk(ws, hs, xc, yc):
        ws = np.asarray(ws, dtype=np.float64)[:, None]
        hs = np.asarray(hs, dtype=np.float64)[:, None]
        return np.hstack([xc - 0.5 * (ws - 1), yc - 0.5 * (hs - 1),
                          xc + 0.5 * (ws - 1), yc + 0.5 * (hs - 1)])

    w, h, xc, yc = whctrs(base)
    size = w * h
    sr = size / np.array(ratios)
    ws = np.round(np.sqrt(sr))
    hs = np.round(ws * np.array(ratios))
    ra = mk(ws, hs, xc, yc)
    out = []
    for i in range(ra.shape[0]):
        w, h, xc, yc = whctrs(ra[i])
        out.append(mk(w * np.array(scales), h * np.array(scales), xc, yc))
    return np.vstack(out).astype(np.float32)


def _grid_anchors(H, W):
    base = _gen_base_anchors()
    sy, sx = np.meshgrid(np.arange(H) * FEAT_STRIDE, np.arange(W) * FEAT_STRIDE,
                         indexing='ij')
    shifts = np.stack([sx.ravel(), sy.ravel(), sx.ravel(), sy.ravel()],
                      axis=1).astype(np.float32)
    return jnp.asarray((shifts[:, None, :] + base[None, :, :]).reshape(-1, 4))


def _decode(anchors, deltas):
    w = anchors[:, 2] - anchors[:, 0] + 1.0
    h = anchors[:, 3] - anchors[:, 1] + 1.0
    cx = anchors[:, 0] + 0.5 * w
    cy = anchors[:, 1] + 0.5 * h
    dx, dy, dw, dh = deltas[:, 0], deltas[:, 1], deltas[:, 2], deltas[:, 3]
    pcx = dx * w + cx
    pcy = dy * h + cy
    pw = jnp.exp(jnp.clip(dw, -10.0, 10.0)) * w
    ph = jnp.exp(jnp.clip(dh, -10.0, 10.0)) * h
    return jnp.stack([pcx - 0.5 * pw, pcy - 0.5 * ph,
                      pcx + 0.5 * pw, pcy + 0.5 * ph], axis=1)


SRC_R = 72
SRC_C = 128
ONE_KEY = 0x3F800000


def _topk_compact_kernel(s_ref, x1_ref, y1_ref, x2_ref, y2_ref, sc_ref):
    s = s_ref[...]
    bits = jax.lax.bitcast_convert_type(s, jnp.int32)
    keys = jnp.where(s < 0.0, jnp.int32(-1), bits)

    def _count_ge(t):
        c = (keys >= t).astype(jnp.float32)
        return jnp.sum(jnp.sum(c, axis=1, keepdims=True), axis=0,
                       keepdims=True)

    def bs_step(_, st):
        lo, hi = st
        mid = (lo + hi + 1) // 2
        ge = _count_ge(mid) >= float(PRE_NMS)
        return (jnp.where(ge, mid, lo), jnp.where(ge, hi, mid - 1))

    lo0 = jnp.full((1, 1), -1, jnp.int32)
    hi0 = jnp.full((1, 1), ONE_KEY, jnp.int32)
    tau, _ = jax.lax.fori_loop(0, 31, bs_step, (lo0, hi0))

    gt = (keys > tau).astype(jnp.float32)
    tie = (keys == tau).astype(jnp.float32)
    ngt = jnp.sum(jnp.sum(gt, axis=1, keepdims=True), axis=0, keepdims=True)
    r_quota = float(PRE_NMS) - ngt

    su = (jax.lax.broadcasted_iota(jnp.int32, (SRC_C, SRC_C), 0) <
          jax.lax.broadcasted_iota(jnp.int32, (SRC_C, SRC_C), 1)
          ).astype(jnp.float32)
    sl = (jax.lax.broadcasted_iota(jnp.int32, (SRC_R, SRC_R), 1) <
          jax.lax.broadcasted_iota(jnp.int32, (SRC_R, SRC_R), 0)
          ).astype(jnp.float32)

    def _excl_prefix(v):
        rowpre = jax.lax.dot_general(
            v, su, (((1,), (0,)), ((), ())),
            preferred_element_type=jnp.float32)
        rowsum = jnp.sum(v, axis=1, keepdims=True)
        offs = jax.lax.dot_general(
            sl, rowsum, (((1,), (0,)), ((), ())),
            preferred_element_type=jnp.float32)
        return rowpre + offs

    tiepre = _excl_prefix(tie)
    m = gt + tie * (tiepre < r_quota).astype(jnp.float32)
    cpos = _excl_prefix(m)

    sc_ref[...] = jnp.zeros((N_PAD, 8), jnp.float32)
    payloads = (x1_ref[...], y1_ref[...], x2_ref[...], y2_ref[...], s)

    def compact_step(t, carry):
        svals = (jax.lax.broadcasted_iota(jnp.int32, (T, 1, 1), 0) +
                 t * T).astype(jnp.float32)
        oh = (cpos[None, :, :] == svals).astype(jnp.float32) * m[None, :, :]
        for c, v in enumerate(payloads):
            acc = jnp.sum(oh * v[None, :, :], axis=2)
            sc_ref[pl.ds(t * T, T), c:c + 1] = jnp.sum(acc, axis=1,
                                                       keepdims=True)
        return carry

    jax.lax.fori_loop(0, K, compact_step, 0)


def _nms_sel_kernel(sc_ref, scT_ref, out_ref, u_ref, p_ref):
    x1c = scT_ref[0:1, :]
    y1c = scT_ref[1:2, :]
    x2c = scT_ref[2:3, :]
    y2c = scT_ref[3:4, :]
    slane = scT_ref[4:5, :]
    areac = (x2c - x1c + 1.0) * (y2c - y1c + 1.0)

    lane_n = jax.lax.broadcasted_iota(jnp.int32, (1, N_PAD), 1)
    sub_t = jax.lax.broadcasted_iota(jnp.int32, (T, 1), 0)

    def build_step(k, carry):
        off = k * T
        bx1 = sc_ref[pl.ds(off, T), 0:1]
        by1 = sc_ref[pl.ds(off, T), 1:2]
        bx2 = sc_ref[pl.ds(off, T), 2:3]
        by2 = sc_ref[pl.ds(off, T), 3:4]
        bs = sc_ref[pl.ds(off, T), 4:5]
        bar = (bx2 - bx1 + 1.0) * (by2 - by1 + 1.0)
        xx1 = jnp.maximum(bx1, x1c)
        yy1 = jnp.maximum(by1, y1c)
        xx2 = jnp.minimum(bx2, x2c)
        yy2 = jnp.minimum(by2, y2c)
        iw = jnp.maximum(xx2 - xx1 + 1.0, 0.0)
        ih = jnp.maximum(yy2 - yy1 + 1.0, 0.0)
        inter = iw * ih
        iou = inter / (bar + areac - inter + 1e-9)
        prec = jnp.where(
            bs > slane, 1.0,
            jnp.where((bs == slane) & ((off + sub_t) < lane_n), 1.0, 0.0))
        p_ref[pl.ds(off, T), :] = prec
        u_ref[pl.ds(off, T), :] = (iou > NMS_THRESH).astype(jnp.float32) * prec
        return carry

    jax.lax.fori_loop(0, K, build_step, 0)
    u = u_ref[...]

    valid = (lane_n < PRE_NMS).astype(jnp.float32)

    def round_cond(state):
        c, _ = state
        return jnp.sum(c) > 0.0

    def round_body(state):
        c, kept = state
        supc = jax.lax.dot_general(
            c, u, (((1,), (0,)), ((), ())),
            preferred_element_type=jnp.float32)
        lead = c * (supc == 0.0).astype(jnp.float32)
        rej = jax.lax.dot_general(
            lead, u, (((1,), (0,)), ((), ())),
            preferred_element_type=jnp.float32)
        kept = kept + lead
        c = c * (1.0 - lead) * (rej == 0.0).astype(jnp.float32)
        return c, kept

    _, keep = jax.lax.while_loop(
        round_cond, round_body,
        (valid, jnp.zeros((1, N_PAD), jnp.float32)))

    kv = keep * valid
    nkv = (1.0 - keep) * valid
    p = p_ref[...]
    both = jnp.concatenate([kv, nkv], axis=0)
    ppre = jax.lax.dot_general(
        both, p, (((1,), (0,)), ((), ())),
        preferred_element_type=jnp.float32)
    kcnt = jnp.sum(kv, axis=1, keepdims=True)
    pos = jnp.where(kv > 0, ppre[0:1, :],
                    jnp.where(nkv > 0, kcnt + ppre[1:2, :], 1e9))

    for t in range(OUT_PAD // T):
        svals = (t * T + sub_t).astype(jnp.float32)
        oh = (pos == svals).astype(jnp.float32)
        for c in range(4):
            col = scT_ref[c:c + 1, :]
            out_ref[pl.ds(t * T, T), c:c + 1] = jnp.sum(
                oh * col, axis=1, keepdims=True)


def _proposal_block(s72, x1, y1, x2, y2):
    sc = pl.pallas_call(
        _topk_compact_kernel,
        out_shape=jax.ShapeDtypeStruct((N_PAD, 8), jnp.float32),
    )(s72, x1, y1, x2, y2)
    scT = jnp.transpose(sc)
    return pl.pallas_call(
        _nms_sel_kernel,
        out_shape=jax.ShapeDtypeStruct((OUT_PAD, 4), jnp.float32),
        scratch_shapes=[
            pltpu.VMEM((N_PAD, N_PAD), jnp.float32),
            pltpu.VMEM((N_PAD, N_PAD), jnp.float32),
        ],
    )(sc, scT)


def kernel(rpn_cls_score, rpn_bbox_pred):
    B, H, W, c2 = rpn_cls_score.shape
    A = c2 // 2
    anchors = _grid_anchors(H, W)

    logits = rpn_cls_score.reshape(B, H, W, A, 2)
    probs = jax.nn.softmax(logits, axis=-1)
    scores = probs[..., 1].reshape(B, -1)
    deltas = rpn_bbox_pred.reshape(B, -1, 4)
    props = jax.vmap(lambda d: _decode(anchors, d))(deltas)
    im_h, im_w = IM_DIMS
    props = jnp.stack([
        jnp.clip(props[..., 0], 0.0, im_w - 1.0),
        jnp.clip(props[..., 1], 0.0, im_h - 1.0),
        jnp.clip(props[..., 2], 0.0, im_w - 1.0),
        jnp.clip(props[..., 3], 0.0, im_h - 1.0)], axis=-1)
    ws = props[..., 2] - props[..., 0] + 1.0
    hs = props[..., 3] - props[..., 1] + 1.0
    ok = (ws >= MIN_SIZE) & (hs >= MIN_SIZE)
    scores = jnp.where(ok, scores, -1e9)

    s72 = scores.reshape(B, SRC_R, SRC_C)
    px1 = props[..., 0].reshape(B, SRC_R, SRC_C)
    py1 = props[..., 1].reshape(B, SRC_R, SRC_C)
    px2 = props[..., 2].reshape(B, SRC_R, SRC_C)
    py2 = props[..., 3].reshape(B, SRC_R, SRC_C)

    sel = jax.vmap(_proposal_block)(s72, px1, py1, px2, py2)[:, :POST_NMS, :]
    bi = jnp.broadcast_to(
        jnp.arange(B, dtype=sel.dtype)[:, None, None], (B, POST_NMS, 1))
    return jnp.concatenate([bi, sel], axis=-1).reshape(B * POST_NMS, 5)
